# baseline (speedup 1.0000x reference)
import jax
import jax.numpy as jnp
from jax import lax
from jax.experimental import pallas as pl
from jax.experimental.pallas import tpu as pltpu

_C = 8
_W = 512


def _sig_kernel(p_ref, o1, o2, o3, o4, s3b_ref, s4b_ref):
    B = p_ref.shape[1]
    f32 = jnp.float32

    lane = lax.broadcasted_iota(jnp.int32, (_C, 3 * _W), 1)
    row = lax.broadcasted_iota(jnp.int32, (_C, 3 * _W), 0)
    e123 = (((lane & 7) == row) & (lane < _W)).astype(f32) \
        + ((((lane >> 3) & 7) == row) & (_W <= lane) & (lane < 2 * _W)).astype(f32) \
        + ((((lane >> 6) & 7) == row) & (2 * _W <= lane)).astype(f32)

    o3[...] = jnp.zeros((B, _W), f32)
    o4[...] = jnp.zeros((B, _C * _W), f32)
    s3b_ref[...] = jnp.zeros((B, _W), f32)
    s4b_ref[...] = jnp.zeros((B, _C * _W), f32)

    def pats(dx):
        d = jnp.dot(dx, e123, preferred_element_type=f32)
        return d[:, :_W], d[:, _W : 2 * _W], d[:, 2 * _W :]

    def chain(d1, d2, d3, s1, s2, s3v):
        ct = s1 + 0.25 * d1
        gt = s2 + (1.0 / 3.0) * (d2 * ct)
        h = s3v + 0.5 * (d3 * gt)
        cv = s1 + (1.0 / 3.0) * d1
        dv = s2 + 0.5 * (d2 * cv)
        s3n = s3v + d3 * dv
        av = s1 + 0.5 * d1
        s2n = s2 + d2 * av
        s1n = s1 + d1
        return h, s1n, s2n, s3n

    half = (p_ref.shape[0] - 1) // 2
    xa1 = p_ref[1]
    xb1 = p_ref[half + 1]
    dxa0 = xa1 - p_ref[0]
    dxb0 = xb1 - p_ref[half]
    da0 = pats(dxa0)
    db0 = pats(dxb0)
    zw = jnp.zeros((B, _W), f32)
    init = (
        xa1, dxa0, da0[0], da0[1], da0[2], zw, zw,
        xb1, dxb0, db0[0], db0[1], db0[2], zw, zw,
    )

    def step(i, carry):
        (xa, dxa, d1a, d2a, d3a, s1a, s2a,
         xb, dxb, d1b, d2b, d3b, s1b, s2b) = carry
        xan = p_ref[i + 2]
        xbn = p_ref[jnp.minimum(half + i + 2, 2 * half)]
        dxan = xan - xa
        dxbn = xbn - xb
        d1an, d2an, d3an = pats(dxan)
        d1bn, d2bn, d3bn = pats(dxbn)

        s3va = o3[...]
        ha, s1a, s2a, s3na = chain(d1a, d2a, d3a, s1a, s2a, s3va)
        o3[...] = s3na
        for j in range(_C):
            o4[:, _W * j : _W * (j + 1)] += dxa[:, j : j + 1] * ha
        s3vb = s3b_ref[...]
        hb, s1b, s2b, s3nb = chain(d1b, d2b, d3b, s1b, s2b, s3vb)
        s3b_ref[...] = s3nb
        for j in range(_C):
            s4b_ref[:, _W * j : _W * (j + 1)] += dxb[:, j : j + 1] * hb

        return (xan, dxan, d1an, d2an, d3an, s1a, s2a,
                xbn, dxbn, d1bn, d2bn, d3bn, s1b, s2b)

    carry = lax.fori_loop(0, half, step, init)
    s1a, s2a = carry[5], carry[6]
    s1b, s2b = carry[12], carry[13]

    s3a = o3[...]
    s3b = s3b_ref[...]
    b1n = s1b[:, :_C]
    b2n = s2b[:, :64]
    pb1_1, pb1_2, pb1_3 = pats(b1n)

    lr8 = lax.broadcasted_iota(jnp.int32, (64, _W), 1)
    rr8 = lax.broadcasted_iota(jnp.int32, (64, _W), 0)
    r8m = ((lr8 >> 3) == rr8).astype(f32)

    o1[...] = s1a + s1b
    o2[...] = s2a + s2b + s1a * pb1_2
    rep8_b2 = jnp.dot(b2n, r8m, preferred_element_type=f32)
    o3[...] = s3a + s3b + s1a * rep8_b2 + s2a * pb1_3
    s1a4 = pltpu.repeat(s1a, _C, axis=1)
    s2a4 = pltpu.repeat(s2a, _C, axis=1)
    s3a4 = pltpu.repeat(s3a, _C, axis=1)
    e3blk = e123[:, 2 * _W :]
    for j in range(_C):
        sl = slice(_W * j, _W * (j + 1))
        rep8_b3_j = jnp.dot(s3b[:, 64 * j : 64 * (j + 1)], r8m,
                            preferred_element_type=f32)
        rep64_b2_j = jnp.dot(b2n[:, _C * j : _C * (j + 1)], e3blk,
                             preferred_element_type=f32)
        o4[:, sl] += (
            s4b_ref[:, sl]
            + s1a4[:, sl] * rep8_b3_j
            + s2a4[:, sl] * rep64_b2_j
            + s3a4[:, sl] * b1n[:, j : j + 1]
        )


def kernel(path):
    n, length, c = path.shape
    pt = jnp.swapaxes(path, 0, 1)
    pt = jnp.concatenate([pt, pt[-1:]], axis=0)
    grid_n = 2
    B = n // grid_n
    out_shape = (
        jax.ShapeDtypeStruct((n, _W), jnp.float32),
        jax.ShapeDtypeStruct((n, _W), jnp.float32),
        jax.ShapeDtypeStruct((n, _W), jnp.float32),
        jax.ShapeDtypeStruct((n, _C * _W), jnp.float32),
    )
    s1t, s2t, s3r, s4r = pl.pallas_call(
        _sig_kernel,
        grid=(grid_n,),
        in_specs=[pl.BlockSpec((length + 1, B, c), lambda i: (0, i, 0))],
        out_specs=(
            pl.BlockSpec((B, _W), lambda i: (i, 0)),
            pl.BlockSpec((B, _W), lambda i: (i, 0)),
            pl.BlockSpec((B, _W), lambda i: (i, 0)),
            pl.BlockSpec((B, _C * _W), lambda i: (i, 0)),
        ),
        out_shape=out_shape,
        scratch_shapes=[
            pltpu.VMEM((B, _W), jnp.float32),
            pltpu.VMEM((B, _C * _W), jnp.float32),
        ],
        compiler_params=pltpu.CompilerParams(
            dimension_semantics=("parallel",),
        ),
        name="signature_scan",
    )(pt)
    s1 = s1t[:, :8]
    s2 = s2t[:, :64].reshape(n, 8, 8).transpose(0, 2, 1).reshape(n, 64)
    s3 = s3r.reshape(n, 8, 8, 8).transpose(0, 3, 2, 1).reshape(n, 512)
    s4 = s4r.reshape(n, 8, 8, 8, 8).transpose(0, 4, 3, 2, 1).reshape(n, 4096)
    return jnp.concatenate([s1, s2, s3, s4], axis=-1)
